# baseline (speedup 1.0000x reference)
import jax
import jax.numpy as jnp
from jax.experimental import pallas as pl

N = 8192
G = 2048
D = 2
C = 3
K = 16
EPS = 1e-06

BN = 256
CH = 8
GL = G // CH

_SORT8 = [(0, 1), (2, 3), (4, 5), (6, 7),
          (0, 2), (1, 3), (4, 6), (5, 7),
          (1, 2), (5, 6),
          (0, 4), (1, 5), (2, 6), (3, 7),
          (2, 4), (3, 5),
          (1, 2), (3, 4), (5, 6)]


def _render_block(x_ref, mus_ref, covs_ref, cols_ref, out_ref):
    x = x_ref[...]
    mu = mus_ref[...]
    cv = covs_ref[...]
    colsp = cols_ref[...]

    x0 = x[:, 0:1]
    x1 = x[:, 1:2]
    dx = x0 - mu[0:1, :]
    dy = x1 - mu[1:2, :]

    c00 = cv[0:1, :]
    c01 = cv[1:2, :]
    c10 = cv[2:3, :]
    c11 = cv[3:4, :]
    inv_det = 1.0 / (c00 * c11 - c01 * c10)
    quad = (c11 * dx * dx - (c01 + c10) * dx * dy + c00 * dy * dy) * inv_det

    qi = jax.lax.bitcast_convert_type(quad, jnp.int32)
    chunks = []
    for c in range(CH):
        qc = (qi[:, c * GL:(c + 1) * GL] & jnp.int32(~7)) | jnp.int32(c)
        chunks.append(jax.lax.bitcast_convert_type(qc, jnp.float32))

    for i, j in _SORT8:
        lo = jnp.minimum(chunks[i], chunks[j])
        hi = jnp.maximum(chunks[i], chunks[j])
        chunks[i], chunks[j] = lo, hi

    qp = chunks[0]
    queue = chunks[1:]
    vs = []
    mms = []
    for _ in range(K):
        v = jnp.min(qp, axis=1, keepdims=True)
        eq = qp == v
        eqf = eq.astype(jnp.float32)
        mms.append(jnp.dot(eqf, colsp, preferred_element_type=jnp.float32))
        qp = jnp.where(eq, queue[0], qp)
        for t in range(CH - 2):
            queue[t] = jnp.where(eq, queue[t + 1], queue[t])
        queue[CH - 2] = jnp.where(eq, jnp.inf, queue[CH - 2])
        vs.append(v)

    V = jnp.concatenate(vs, axis=1)
    EV = jnp.exp(-0.5 * V)
    CID = jax.lax.bitcast_convert_type(V, jnp.int32) & jnp.int32(7)

    den = jnp.full((BN, 1), EPS, jnp.float32)
    num = jnp.zeros((BN, C), jnp.float32)
    used = jnp.zeros((BN, 1), jnp.float32)
    for r in range(K):
        mm = mms[r]
        cnt = mm[:, C * CH:C * CH + 1]
        allowed = jnp.minimum(cnt, float(K) - used)
        used = used + allowed
        ev = EV[:, r:r + 1]
        den = den + allowed * ev
        scale = (allowed / cnt) * ev
        cid = CID[:, r:r + 1]
        csel = jnp.zeros((BN, C), jnp.float32)
        for c in range(CH):
            hit = (cid == c).astype(jnp.float32)
            csel = csel + hit * mm[:, C * c:C * (c + 1)]
        num = num + scale * csel
    out_ref[...] = num / den


@jax.jit
def kernel(x, mus, covs, cols):
    mus_t = mus[0].T
    covs4 = covs[0].reshape(G, 4).T
    colsp = jnp.concatenate(
        [cols[0][c * GL:(c + 1) * GL] for c in range(CH)]
        + [jnp.ones((GL, 1), jnp.float32)], axis=1)
    grid = (N // BN,)
    out = pl.pallas_call(
        _render_block,
        grid=grid,
        in_specs=[
            pl.BlockSpec((BN, D), lambda i: (i, 0)),
            pl.BlockSpec((D, G), lambda i: (0, 0)),
            pl.BlockSpec((4, G), lambda i: (0, 0)),
            pl.BlockSpec((GL, C * CH + 1), lambda i: (0, 0)),
        ],
        out_specs=pl.BlockSpec((BN, C), lambda i: (i, 0)),
        out_shape=jax.ShapeDtypeStruct((N, C), jnp.float32),
    )(x, mus_t, covs4, colsp)
    return out


# speedup vs baseline: 2.1438x; 2.1438x over previous
import jax
import jax.numpy as jnp
from jax.experimental import pallas as pl

N = 8192
G = 2048
D = 2
C = 3
K = 16
EPS = 1e-06

BN = 512


def _render_block(x_ref, mus_ref, covs_ref, cols_ref, out_ref):
    x = x_ref[...]
    mu = mus_ref[...]
    cv = covs_ref[...]
    colsp = cols_ref[...]

    x0 = x[:, 0:1]
    x1 = x[:, 1:2]
    dx = x0 - mu[0:1, :]
    dy = x1 - mu[1:2, :]

    c00 = cv[0:1, :]
    c01 = cv[1:2, :]
    c10 = cv[2:3, :]
    c11 = cv[3:4, :]
    inv_det = 1.0 / (c00 * c11 - c01 * c10)
    quad = (c11 * dx * dx - (c01 + c10) * dx * dy + c00 * dy * dy) * inv_det

    q = quad
    vs = []
    mms = []
    for _ in range(K):
        v = jnp.min(q, axis=1, keepdims=True)
        eq = q == v
        eqf = eq.astype(jnp.float32)
        mms.append(jnp.dot(eqf, colsp, preferred_element_type=jnp.float32))
        q = jnp.where(eq, jnp.inf, q)
        vs.append(v)

    V = jnp.concatenate(vs, axis=1)
    EV = jnp.exp(-0.5 * V)
    den = jnp.full((BN, 1), EPS, jnp.float32)
    num = jnp.zeros((BN, C), jnp.float32)
    used = jnp.zeros((BN, 1), jnp.float32)
    for r in range(K):
        cnt = mms[r][:, C:C + 1]
        allowed = jnp.minimum(cnt, float(K) - used)
        used = used + allowed
        ev = EV[:, r:r + 1]
        den = den + allowed * ev
        num = num + ((allowed / cnt) * ev) * mms[r][:, 0:C]
    out_ref[...] = num / den


@jax.jit
def kernel(x, mus, covs, cols):
    mus_t = mus[0].T
    covs4 = covs[0].reshape(G, 4).T
    colsp = jnp.concatenate(
        [cols[0], jnp.ones((G, 1), jnp.float32)], axis=1)
    grid = (N // BN,)
    out = pl.pallas_call(
        _render_block,
        grid=grid,
        in_specs=[
            pl.BlockSpec((BN, D), lambda i: (i, 0)),
            pl.BlockSpec((D, G), lambda i: (0, 0)),
            pl.BlockSpec((4, G), lambda i: (0, 0)),
            pl.BlockSpec((G, C + 1), lambda i: (0, 0)),
        ],
        out_specs=pl.BlockSpec((BN, C), lambda i: (i, 0)),
        out_shape=jax.ShapeDtypeStruct((N, C), jnp.float32),
    )(x, mus_t, covs4, colsp)
    return out
